# baseline (device time: 56520 ns/iter reference)
import jax
import jax.numpy as jnp
from jax import lax
from jax.experimental import pallas as pl
from jax.experimental.pallas import tpu as pltpu


def kernel(partial, resid, gamma):
    m, d = resid.shape
    gamma2d = gamma.reshape(1, d)

    def body(partial_ref, resid_ref, gamma_ref, out_ref,
             peer_ref, send_sem, recv_sem):
        my_x = lax.axis_index("x")
        my_y = lax.axis_index("y")
        my_z = lax.axis_index("z")
        peer = (1 - my_x, my_y, my_z)

        barrier_sem = pltpu.get_barrier_semaphore()
        pl.semaphore_signal(
            barrier_sem, inc=1,
            device_id=peer, device_id_type=pl.DeviceIdType.MESH,
        )
        pl.semaphore_wait(barrier_sem, 1)

        rdma = pltpu.make_async_remote_copy(
            src_ref=partial_ref,
            dst_ref=peer_ref,
            send_sem=send_sem,
            recv_sem=recv_sem,
            device_id=peer,
            device_id_type=pl.DeviceIdType.MESH,
        )
        rdma.start()
        rdma.wait()

        y = partial_ref[0] + peer_ref[0] + resid_ref[...]
        ms = jnp.mean(y * y, axis=-1, keepdims=True)
        out_ref[...] = y * lax.rsqrt(ms + 1e-6) * gamma_ref[...]

    return pl.pallas_call(
        body,
        out_shape=jax.ShapeDtypeStruct((m, d), jnp.float32),
        in_specs=[
            pl.BlockSpec(memory_space=pltpu.VMEM),
            pl.BlockSpec(memory_space=pltpu.VMEM),
            pl.BlockSpec(memory_space=pltpu.VMEM),
        ],
        out_specs=pl.BlockSpec(memory_space=pltpu.VMEM),
        scratch_shapes=[
            pltpu.VMEM(partial.shape, jnp.float32),
            pltpu.SemaphoreType.DMA,
            pltpu.SemaphoreType.DMA,
        ],
        compiler_params=pltpu.CompilerParams(collective_id=0),
    )(partial, resid, gamma2d)


# device time: 39977 ns/iter; 1.4138x vs baseline; 1.4138x over previous
import jax
import jax.numpy as jnp
from jax import lax
from jax.experimental import pallas as pl
from jax.experimental.pallas import tpu as pltpu

M = 1024
D = 1024
NCH = 8
CH = M // NCH
EPS = 1e-6
_MESH = pl.DeviceIdType.MESH


def kernel(partial, resid, gamma):
    gamma2d = gamma.reshape(1, D)

    def body(partial_ref, resid_ref, gamma_ref, out_ref,
             xrecv, own_buf, stage, a_recv, b_recv, twin_recv,
             send_sems, recv_sems):
        x = lax.axis_index("x")
        y = lax.axis_index("y")
        z = lax.axis_index("z")
        p = jnp.where(y == 0, z, 7 - z)
        c_own = (p + 4 * x) % 8
        c_twin = (c_own + 4) % 8

        def plane_coords(q):
            return (q // 4, jnp.where(q < 4, q, 7 - q))

        twin = (1 - x, y, z)
        ny, nz = plane_coords((p + 1) % 8)
        nxt = (x, ny, nz)
        py, pz = plane_coords((p + 7) % 8)
        prv = (x, py, pz)

        def rdma(src, dst, sem_idx, dev):
            return pltpu.make_async_remote_copy(
                src_ref=src, dst_ref=dst,
                send_sem=send_sems.at[sem_idx],
                recv_sem=recv_sems.at[sem_idx],
                device_id=dev, device_id_type=_MESH,
            )

        barrier = pltpu.get_barrier_semaphore()
        for nbr in (twin, nxt, prv):
            pl.semaphore_signal(barrier, inc=1, device_id=nbr,
                                device_id_type=_MESH)
        pl.semaphore_wait(barrier, 3)

        stage[...] = partial_ref[0, pl.ds(c_twin * CH, CH), :]
        x1 = rdma(stage, xrecv, 0, twin)
        x1.start()
        x1.wait_recv()

        rows_own = pl.ds(c_own * CH, CH)
        ysum = partial_ref[0, rows_own, :] + xrecv[...] + resid_ref[rows_own, :]
        ms = jnp.mean(ysum * ysum, axis=-1, keepdims=True)
        normed = ysum * lax.rsqrt(ms + EPS) * gamma_ref[...]
        own_buf[...] = normed
        out_ref[rows_own, :] = normed

        def store(chunk_idx, buf):
            out_ref[pl.ds((chunk_idx % 8) * CH, CH), :] = buf[...]

        a0 = rdma(own_buf, a_recv.at[0], 1, nxt)
        b0 = rdma(own_buf, b_recv.at[0], 4, prv)
        tw = rdma(own_buf, twin_recv, 7, twin)
        a0.start()
        b0.start()
        tw.start()

        a0.wait_recv()
        a1 = rdma(a_recv.at[0], a_recv.at[1], 2, nxt)
        a1.start()
        b0.wait_recv()
        b1 = rdma(b_recv.at[0], b_recv.at[1], 5, prv)
        b1.start()
        store(c_own + 7, a_recv.at[0])
        store(c_own + 1, b_recv.at[0])

        tw.wait_recv()
        store(c_own + 4, twin_recv)

        a1.wait_recv()
        a2 = rdma(a_recv.at[1], a_recv.at[2], 3, nxt)
        a2.start()
        b1.wait_recv()
        b2 = rdma(b_recv.at[1], b_recv.at[2], 6, prv)
        b2.start()
        store(c_own + 6, a_recv.at[1])
        store(c_own + 2, b_recv.at[1])

        a2.wait_recv()
        store(c_own + 5, a_recv.at[2])
        b2.wait_recv()
        store(c_own + 3, b_recv.at[2])

        for d in (x1, a0, b0, tw, a1, b1, a2, b2):
            d.wait_send()

    return pl.pallas_call(
        body,
        out_shape=jax.ShapeDtypeStruct((M, D), jnp.float32),
        in_specs=[
            pl.BlockSpec(memory_space=pltpu.VMEM),
            pl.BlockSpec(memory_space=pltpu.VMEM),
            pl.BlockSpec(memory_space=pltpu.VMEM),
        ],
        out_specs=pl.BlockSpec(memory_space=pltpu.VMEM),
        scratch_shapes=[
            pltpu.VMEM((CH, D), jnp.float32),
            pltpu.VMEM((CH, D), jnp.float32),
            pltpu.VMEM((CH, D), jnp.float32),
            pltpu.VMEM((3, CH, D), jnp.float32),
            pltpu.VMEM((3, CH, D), jnp.float32),
            pltpu.VMEM((CH, D), jnp.float32),
            pltpu.SemaphoreType.DMA((8,)),
            pltpu.SemaphoreType.DMA((8,)),
        ],
        compiler_params=pltpu.CompilerParams(collective_id=0),
    )(partial, resid, gamma2d)


# device time: 32890 ns/iter; 1.7185x vs baseline; 1.2155x over previous
import jax
import jax.numpy as jnp
from jax import lax
from jax.experimental import pallas as pl
from jax.experimental.pallas import tpu as pltpu

M = 1024
D = 1024
NCH = 8
CH = M // NCH
SUB = CH // 2
EPS = 1e-6
_MESH = pl.DeviceIdType.MESH


def kernel(partial, resid, gamma):
    gamma2d = gamma.reshape(1, D)

    def body(partial_ref, resid_ref, gamma_ref, out_ref,
             xrecv, stage, send_sems, recv_sems):
        x = lax.axis_index("x")
        y = lax.axis_index("y")
        z = lax.axis_index("z")
        p = jnp.where(y == 0, z, 7 - z)
        c_own = (p + 4 * x) % 8
        c_twin = (c_own + 4) % 8

        def plane_coords(q):
            return (q // 4, jnp.where(q < 4, q, 7 - q))

        twin = (1 - x, y, z)
        ny, nz = plane_coords((p + 1) % 8)
        nxt = (x, ny, nz)
        py, pz = plane_coords((p + 7) % 8)
        prv = (x, py, pz)

        def orows(chunk, j):
            return out_ref.at[pl.ds((chunk % 8) * CH + j * SUB, SUB), :]

        def rdma(src, dst, sem_idx, dev):
            return pltpu.make_async_remote_copy(
                src_ref=src, dst_ref=dst,
                send_sem=send_sems.at[sem_idx],
                recv_sem=recv_sems.at[sem_idx],
                device_id=dev, device_id_type=_MESH,
            )

        barrier = pltpu.get_barrier_semaphore()
        for nbr in (twin, nxt, prv):
            pl.semaphore_signal(barrier, inc=1, device_id=nbr,
                                device_id_type=_MESH)
        pl.semaphore_wait(barrier, 3)

        stage[...] = partial_ref[0, pl.ds(c_twin * CH, CH), :]
        x1 = [rdma(stage.at[pl.ds(j * SUB, SUB), :],
                   xrecv.at[pl.ds(j * SUB, SUB), :], j, twin)
              for j in range(2)]
        x1[0].start()
        x1[1].start()

        a = [[None, None] for _ in range(3)]
        b = [[None, None] for _ in range(3)]
        t = [None, None]

        for j in range(2):
            x1[j].wait_recv()
            rows_oj = pl.ds(c_own * CH + j * SUB, SUB)
            ysum = (partial_ref[0, rows_oj, :]
                    + xrecv[pl.ds(j * SUB, SUB), :]
                    + resid_ref[rows_oj, :])
            ms = jnp.mean(ysum * ysum, axis=-1, keepdims=True)
            out_ref[rows_oj, :] = ysum * lax.rsqrt(ms + EPS) * gamma_ref[...]

            a[0][j] = rdma(orows(c_own, j), orows(c_own, j), 2 + j, nxt)
            b[0][j] = rdma(orows(c_own, j), orows(c_own, j), 8 + j, prv)
            t[j] = rdma(orows(c_own, j), orows(c_own, j), 14 + j, twin)
            a[0][j].start()
            b[0][j].start()
            t[j].start()

        for s in range(2):
            ca = c_own + 7 - s
            cb = c_own + 1 + s
            for j in range(2):
                a[s][j].wait_recv()
                a[s + 1][j] = rdma(orows(ca, j), orows(ca, j),
                                   2 + 2 * (s + 1) + j, nxt)
                a[s + 1][j].start()
                b[s][j].wait_recv()
                b[s + 1][j] = rdma(orows(cb, j), orows(cb, j),
                                   8 + 2 * (s + 1) + j, prv)
                b[s + 1][j].start()

        for j in range(2):
            a[2][j].wait_recv()
            b[2][j].wait_recv()
            t[j].wait_recv()

        for d in x1 + [d for row in a for d in row] \
                + [d for row in b for d in row] + t:
            d.wait_send()

    return pl.pallas_call(
        body,
        out_shape=jax.ShapeDtypeStruct((M, D), jnp.float32),
        in_specs=[
            pl.BlockSpec(memory_space=pltpu.VMEM),
            pl.BlockSpec(memory_space=pltpu.VMEM),
            pl.BlockSpec(memory_space=pltpu.VMEM),
        ],
        out_specs=pl.BlockSpec(memory_space=pltpu.VMEM),
        scratch_shapes=[
            pltpu.VMEM((CH, D), jnp.float32),
            pltpu.VMEM((CH, D), jnp.float32),
            pltpu.SemaphoreType.DMA((16,)),
            pltpu.SemaphoreType.DMA((16,)),
        ],
        compiler_params=pltpu.CompilerParams(collective_id=0),
    )(partial, resid, gamma2d)
